# baseline (speedup 1.0000x reference)
import jax
import jax.numpy as jnp
from jax import lax
from jax.experimental import pallas as pl
from jax.experimental.pallas import tpu as pltpu
from jax.experimental.pallas import tpu_sc as plsc

NC = 2
NS = 16
CW = 8
C = 256
NB = 4


def _sc_aggregate(xs2, srcx, dstw, n_pad, dh, nslots):
  rows_per_sub = n_pad // NS
  H = C // 2

  zeros_blk = jnp.zeros((rows_per_sub, dh), jnp.float32)
  zeros_cnt = jnp.zeros((rows_per_sub, CW), jnp.float32)
  ones_blk = jnp.ones((H, CW), jnp.float32)

  mesh = plsc.VectorSubcoreMesh(core_axis_name="c", subcore_axis_name="s",
                                num_cores=NC, num_subcores=NS)

  assert nslots > 2 * NB and (nslots - NB - 1) % NB == 0
  loop_iters = (nslots - NB - 1) // NB

  def body(x_hbm, src_hbm, dst_hbm, zf_hbm, zc_hbm, ones_hbm,
           sum_out, cnt_out, acc, cnt_sh, ones_v, *bufs):
    cid = lax.axis_index("c")
    sid = lax.axis_index("s")
    base_n = sid * rows_per_sub
    svs, dvs, rvs, gs, ss = (bufs[0:NB], bufs[NB:2 * NB], bufs[2 * NB:3 * NB],
                             bufs[3 * NB:4 * NB], bufs[4 * NB:5 * NB])

    def start_chunk(i, b):
      pltpu.sync_copy(src_hbm.at[cid, sid, pl.ds(2 * i, 2)], svs[b])
      pltpu.sync_copy(dst_hbm.at[sid, pl.ds(2 * i, 2)], dvs[b])
      pltpu.async_copy(x_hbm.at[svs[b].at[0]], rvs[b].at[pl.ds(0, H)], gs[b])
      pltpu.async_copy(x_hbm.at[svs[b].at[1]], rvs[b].at[pl.ds(H, H)], gs[b])

    def wait_gather(b):
      pltpu.make_async_copy(
          x_hbm.at[svs[b].at[0]], rvs[b].at[pl.ds(0, H)], gs[b]).wait()
      pltpu.make_async_copy(
          x_hbm.at[svs[b].at[1]], rvs[b].at[pl.ds(H, H)], gs[b]).wait()

    def start_scatter(b):
      pltpu.async_copy(rvs[b].at[pl.ds(0, H)], acc.at[dvs[b].at[0]],
                       ss[b], add=True)
      pltpu.async_copy(rvs[b].at[pl.ds(H, H)], acc.at[dvs[b].at[1]],
                       ss[b], add=True)

      @pl.when(cid == 0)
      def _():
        pltpu.async_copy(ones_v, cnt_sh.at[dvs[b].at[0]], ss[b], add=True)
        pltpu.async_copy(ones_v, cnt_sh.at[dvs[b].at[1]], ss[b], add=True)

    def wait_scatter(b):
      pltpu.make_async_copy(rvs[b].at[pl.ds(0, H)], acc.at[dvs[b].at[0]],
                            ss[b]).wait()
      pltpu.make_async_copy(rvs[b].at[pl.ds(H, H)], acc.at[dvs[b].at[1]],
                            ss[b]).wait()

      @pl.when(cid == 0)
      def _():
        pltpu.make_async_copy(ones_v, cnt_sh.at[dvs[b].at[0]], ss[b]).wait()
        pltpu.make_async_copy(ones_v, cnt_sh.at[dvs[b].at[1]], ss[b]).wait()

    pltpu.sync_copy(ones_hbm, ones_v)
    for b in range(NB):
      start_chunk(b, b)
    pltpu.sync_copy(zf_hbm, acc.at[pl.ds(base_n, rows_per_sub)])
    pltpu.sync_copy(zc_hbm, cnt_sh.at[pl.ds(base_n, rows_per_sub)])
    plsc.subcore_barrier()
    for b in range(NB - 1):
      wait_gather(b)
      start_scatter(b)

    def chunk_group(k, carry):
      i0 = NB + NB * k
      for b in range(NB):
        wait_scatter(b)
        start_chunk(i0 + b, b)
        wait_gather((b - 1) % NB)
        start_scatter((b - 1) % NB)
      return carry

    lax.fori_loop(0, loop_iters, chunk_group, 0)
    last = nslots - 1
    b = last % NB
    wait_scatter(b)
    start_chunk(last, b)
    wait_gather((b - 1) % NB)
    start_scatter((b - 1) % NB)
    wait_gather(b)
    start_scatter(b)
    for bb in range(NB):
      wait_scatter(bb)
    plsc.subcore_barrier()
    pltpu.sync_copy(acc.at[pl.ds(base_n, rows_per_sub)],
                    sum_out.at[cid, pl.ds(base_n, rows_per_sub)])

    @pl.when(cid == 0)
    def _():
      pltpu.sync_copy(cnt_sh.at[pl.ds(base_n, rows_per_sub)],
                      cnt_out.at[pl.ds(base_n, rows_per_sub)])

  call = pl.kernel(
      body,
      out_type=(
          jax.ShapeDtypeStruct((NC, n_pad, dh), jnp.float32),
          jax.ShapeDtypeStruct((n_pad, CW), jnp.float32),
      ),
      mesh=mesh,
      scratch_types=(
          [
              pltpu.VMEM_SHARED((n_pad, dh), jnp.float32),
              pltpu.VMEM_SHARED((n_pad, CW), jnp.float32),
              pltpu.VMEM((H, CW), jnp.float32),
          ]
          + [pltpu.VMEM((2, H), jnp.int32) for _ in range(2 * NB)]
          + [pltpu.VMEM((C, dh), jnp.float32) for _ in range(NB)]
          + [pltpu.SemaphoreType.DMA for _ in range(2 * NB)]
      ),
      compiler_params=pltpu.CompilerParams(use_tc_tiling_on_sc=False),
  )
  return call(xs2, srcx, dstw, zeros_blk, zeros_cnt, ones_blk)


def _tc_mlp(x, sums, cnt, W_l, W_r, W1, W2, W3, b_l, b1, b2, b3):
  n, d = x.shape
  dh = d // 2
  out_dim = W3.shape[0]
  R = 1000
  assert n % R == 0
  grid = n // R
  dn = (((1,), (1,)), ((), ()))

  def body(xb, s0b, s1b, cb, wl, wr, w1, w2, w3, bl, bb1, bb2, bb3, ob):
    summed = jnp.concatenate([s0b[0], s1b[0]], axis=1)
    counts = cb[:, :1]
    mean = summed / jnp.maximum(counts, 1.0)
    f32 = jnp.float32
    h = (lax.dot_general(mean, wl[...], dn, preferred_element_type=f32)
         + lax.dot_general(xb[...], wr[...], dn, preferred_element_type=f32)
         + bl[...])
    h1 = jnp.maximum(
        lax.dot_general(h, w1[...], dn, preferred_element_type=f32)
        + bb1[...], 0.0)
    h2 = jnp.maximum(
        lax.dot_general(h1, w2[...], dn, preferred_element_type=f32)
        + bb2[...], 0.0)
    ob[...] = (lax.dot_general(h2, w3[...], dn, preferred_element_type=f32)
               + bb3[...])

  row_spec = lambda c: pl.BlockSpec((R, c), lambda i: (i, 0))
  sum_spec = lambda k: pl.BlockSpec((1, R, dh), lambda i, _k=k: (_k, i, 0))
  full_spec = lambda r, c: pl.BlockSpec((r, c), lambda i: (0, 0))
  return pl.pallas_call(
      body,
      grid=(grid,),
      in_specs=[
          row_spec(d), sum_spec(0), sum_spec(1), row_spec(CW),
          full_spec(*W_l.shape), full_spec(*W_r.shape),
          full_spec(*W1.shape), full_spec(*W2.shape), full_spec(*W3.shape),
          full_spec(*b_l.shape), full_spec(*b1.shape),
          full_spec(*b2.shape), full_spec(*b3.shape),
      ],
      out_specs=row_spec(out_dim),
      out_shape=jax.ShapeDtypeStruct((n, out_dim), jnp.float32),
  )(x, sums, sums, cnt, W_l, W_r, W1, W2, W3, b_l, b1, b2, b3)


@jax.jit
def kernel(x, edge_index, W_l, b_l, W_r, W1, b1, W2, b2, W3, b3):
  n, d = x.shape
  dh = d // 2
  e = edge_index.shape[1]
  src = edge_index[0]
  dst = edge_index[1]
  n_pad = ((n + NS * 8 - 1) // (NS * 8)) * (NS * 8)
  xs2 = jnp.concatenate([x[:, :dh], x[:, dh:]], 0)
  assert e % NS == 0
  e_tile = e // NS
  nslots = (e_tile + C - 1) // C
  while nslots % NB != 1:
    nslots += 1
  pad = nslots * C - e_tile
  src_t = jnp.concatenate(
      [src.reshape(NS, e_tile), jnp.zeros((NS, pad), jnp.int32)], 1)
  dst_t = jnp.concatenate(
      [dst.reshape(NS, e_tile), jnp.full((NS, pad), n, jnp.int32)], 1)
  src_t = src_t.reshape(NS, 2 * nslots, 128)
  dstw = dst_t.reshape(NS, 2 * nslots, 128)
  srcx = jnp.stack([src_t, src_t + n])
  sums, cnt = _sc_aggregate(xs2, srcx, dstw, n_pad, dh, nslots)
  return _tc_mlp(
      x, sums, cnt, W_l, W_r, W1, W2, W3,
      b_l.reshape(1, -1), b1.reshape(1, -1), b2.reshape(1, -1),
      b3.reshape(1, -1))


# speedup vs baseline: 2.4361x; 2.4361x over previous
import jax
import jax.numpy as jnp
from jax import lax
from jax.experimental import pallas as pl
from jax.experimental.pallas import tpu as pltpu
from jax.experimental.pallas import tpu_sc as plsc

NC = 2
NS = 16
NW = NC * NS
CW = 8


def _sc_aggregate(x, edge_index, n_pad):
  e = edge_index.shape[1]
  d = x.shape[1]
  assert e % NW == 0
  e_per_w = e // NW
  C = 80
  assert e_per_w % C == 0
  nchunks = e_per_w // C
  rows_per_sub = n_pad // NS
  NB = 4
  eiw = edge_index.reshape(2, NW * nchunks, C).transpose(1, 0, 2)

  zeros_blk = jnp.zeros((rows_per_sub, d), jnp.float32)
  zeros_cnt = jnp.zeros((rows_per_sub, CW), jnp.float32)
  ones_blk = jnp.ones((C, CW), jnp.float32)

  mesh = plsc.VectorSubcoreMesh(core_axis_name="c", subcore_axis_name="s",
                                num_cores=NC, num_subcores=NS)

  assert nchunks > 2 * NB and (nchunks - NB - 1) % NB == 0
  loop_iters = (nchunks - NB - 1) // NB

  def body(x_hbm, ei_hbm, zf_hbm, zc_hbm, ones_hbm,
           sum_out, cnt0_out, cnt1_out, acc, cnt_sh, ones_v, *bufs):
    cid = lax.axis_index("c")
    sid = lax.axis_index("s")
    wid = cid * NS + sid
    base_n = sid * rows_per_sub
    base_c = wid * nchunks
    ivs, rvs, gs, ss, isems = (bufs[0:NB], bufs[NB:2 * NB],
                               bufs[2 * NB:3 * NB], bufs[3 * NB:4 * NB],
                               bufs[4 * NB:5 * NB])

    def start_idx(i, b):
      pltpu.async_copy(ei_hbm.at[base_c + i], ivs[b], isems[b])

    def wait_idx(i, b):
      pltpu.make_async_copy(ei_hbm.at[base_c + i], ivs[b], isems[b]).wait()

    def start_gather(b):
      pltpu.async_copy(x_hbm.at[ivs[b].at[0]], rvs[b], gs[b])

    def wait_gather(b):
      pltpu.make_async_copy(x_hbm.at[ivs[b].at[0]], rvs[b], gs[b]).wait()

    def start_scatter(b):
      pltpu.async_copy(rvs[b], acc.at[ivs[b].at[1]], ss[b], add=True)
      pltpu.async_copy(ones_v, cnt_sh.at[ivs[b].at[1]], ss[b], add=True)

    def wait_scatter(b):
      pltpu.make_async_copy(rvs[b], acc.at[ivs[b].at[1]], ss[b]).wait()
      pltpu.make_async_copy(ones_v, cnt_sh.at[ivs[b].at[1]], ss[b]).wait()

    pltpu.sync_copy(ones_hbm, ones_v)
    for b in range(NB - 1):
      start_idx(b, b)
    pltpu.sync_copy(zf_hbm, acc.at[pl.ds(base_n, rows_per_sub)])
    pltpu.sync_copy(zc_hbm, cnt_sh.at[pl.ds(base_n, rows_per_sub)])
    wait_idx(0, 0)
    start_gather(0)
    wait_idx(1, 1)
    start_gather(1)
    plsc.subcore_barrier()
    wait_gather(0)
    start_scatter(0)
    start_idx(3, 3)
    wait_idx(2, 2)
    start_gather(2)
    wait_gather(1)
    start_scatter(1)

    def slot(i, b):
      bn = (b + 1) % NB
      wait_scatter(bn)
      start_idx(i + 1, bn)
      wait_idx(i, b)
      start_gather(b)
      wait_gather((b - 1) % NB)
      start_scatter((b - 1) % NB)

    def slot_group(k, carry):
      i0 = 3 + NB * k
      for j in range(NB):
        slot(i0 + j, (3 + j) % NB)
      return carry

    lax.fori_loop(0, loop_iters, slot_group, 0)
    i = nchunks - 2
    b = i % NB
    wait_scatter((b + 1) % NB)
    start_idx(i + 1, (b + 1) % NB)
    wait_idx(i, b)
    start_gather(b)
    wait_gather((b - 1) % NB)
    start_scatter((b - 1) % NB)
    i = nchunks - 1
    b = i % NB
    wait_idx(i, b)
    start_gather(b)
    wait_gather((b - 1) % NB)
    start_scatter((b - 1) % NB)
    wait_gather(b)
    start_scatter(b)
    for bb in range(NB):
      wait_scatter(bb)
    plsc.subcore_barrier()
    pltpu.sync_copy(acc.at[pl.ds(base_n, rows_per_sub)],
                    sum_out.at[cid, pl.ds(base_n, rows_per_sub)])

    @pl.when(cid == 0)
    def _():
      pltpu.sync_copy(cnt_sh.at[pl.ds(base_n, rows_per_sub)],
                      cnt0_out.at[pl.ds(base_n, rows_per_sub)])

    @pl.when(cid == 1)
    def _():
      pltpu.sync_copy(cnt_sh.at[pl.ds(base_n, rows_per_sub)],
                      cnt1_out.at[pl.ds(base_n, rows_per_sub)])

  call = pl.kernel(
      body,
      out_type=(
          jax.ShapeDtypeStruct((NC, n_pad, d), jnp.float32),
          jax.ShapeDtypeStruct((n_pad, CW), jnp.float32),
          jax.ShapeDtypeStruct((n_pad, CW), jnp.float32),
      ),
      mesh=mesh,
      scratch_types=(
          [
              pltpu.VMEM_SHARED((n_pad, d), jnp.float32),
              pltpu.VMEM_SHARED((n_pad, CW), jnp.float32),
              pltpu.VMEM((C, CW), jnp.float32),
          ]
          + [pltpu.VMEM((2, C), jnp.int32) for _ in range(NB)]
          + [pltpu.VMEM((C, d), jnp.float32) for _ in range(NB)]
          + [pltpu.SemaphoreType.DMA for _ in range(3 * NB)]
      ),
      compiler_params=pltpu.CompilerParams(use_tc_tiling_on_sc=False),
  )
  return call(x, eiw, zeros_blk, zeros_cnt, ones_blk)


def _tc_mlp(x, sums, cnt0, cnt1, W_l, W_r, W1, W2, W3, b_l, b1, b2, b3):
  n, d = x.shape
  out_dim = W3.shape[0]
  R = 1000
  assert n % R == 0
  grid = n // R
  dn = (((1,), (1,)), ((), ()))

  def body(xb, s0b, s1b, c0b, c1b, wl, wr, w1, w2, w3,
           bl, bb1, bb2, bb3, ob):
    summed = s0b[0] + s1b[0]
    counts = c0b[:, :1] + c1b[:, :1]
    mean = summed / jnp.maximum(counts, 1.0)
    f32 = jnp.float32
    h = (lax.dot_general(mean, wl[...], dn, preferred_element_type=f32)
         + lax.dot_general(xb[...], wr[...], dn, preferred_element_type=f32)
         + bl[...])
    h1 = jnp.maximum(
        lax.dot_general(h, w1[...], dn, preferred_element_type=f32)
        + bb1[...], 0.0)
    h2 = jnp.maximum(
        lax.dot_general(h1, w2[...], dn, preferred_element_type=f32)
        + bb2[...], 0.0)
    ob[...] = (lax.dot_general(h2, w3[...], dn, preferred_element_type=f32)
               + bb3[...])

  row_spec = lambda c: pl.BlockSpec((R, c), lambda i: (i, 0))
  sum_spec = lambda k: pl.BlockSpec((1, R, d), lambda i, _k=k: (_k, i, 0))
  full_spec = lambda r, c: pl.BlockSpec((r, c), lambda i: (0, 0))
  return pl.pallas_call(
      body,
      grid=(grid,),
      in_specs=[
          row_spec(d), sum_spec(0), sum_spec(1), row_spec(CW), row_spec(CW),
          full_spec(*W_l.shape), full_spec(*W_r.shape),
          full_spec(*W1.shape), full_spec(*W2.shape), full_spec(*W3.shape),
          full_spec(*b_l.shape), full_spec(*b1.shape),
          full_spec(*b2.shape), full_spec(*b3.shape),
      ],
      out_specs=row_spec(out_dim),
      out_shape=jax.ShapeDtypeStruct((n, out_dim), jnp.float32),
  )(x, sums, sums, cnt0, cnt1, W_l, W_r, W1, W2, W3, b_l, b1, b2, b3)


@jax.jit
def kernel(x, edge_index, W_l, b_l, W_r, W1, b1, W2, b2, W3, b3):
  n, d = x.shape
  n_pad = ((n + NS * 8 - 1) // (NS * 8)) * (NS * 8)
  sums, cnt0, cnt1 = _sc_aggregate(x, edge_index, n_pad)
  return _tc_mlp(
      x, sums, cnt0, cnt1, W_l, W_r, W1, W2, W3,
      b_l.reshape(1, -1), b1.reshape(1, -1), b2.reshape(1, -1),
      b3.reshape(1, -1))


# speedup vs baseline: 2.4835x; 1.0194x over previous
import jax
import jax.numpy as jnp
from jax import lax
from jax.experimental import pallas as pl
from jax.experimental.pallas import tpu as pltpu
from jax.experimental.pallas import tpu_sc as plsc

NC = 2
NS = 16
NW = NC * NS
CW = 8


def _sc_aggregate(x, edge_index, n_pad):
  e = edge_index.shape[1]
  d = x.shape[1]
  assert e % NW == 0
  e_per_w = e // NW
  C = 80
  assert e_per_w % C == 0
  nchunks = e_per_w // C
  rows_per_sub = n_pad // NS
  NB = 4
  eiw = edge_index.reshape(2, NW * nchunks, C).transpose(1, 0, 2)

  zeros_blk = jnp.zeros((rows_per_sub, d), jnp.float32)
  zeros_cnt = jnp.zeros((rows_per_sub, CW), jnp.float32)
  ones_blk = jnp.ones((C, CW), jnp.float32)

  mesh = plsc.VectorSubcoreMesh(core_axis_name="c", subcore_axis_name="s",
                                num_cores=NC, num_subcores=NS)

  assert nchunks > 2 * NB and (nchunks - NB - 1) % NB == 0
  loop_iters = (nchunks - NB - 1) // NB

  def body(x_hbm, ei_hbm, zf_hbm, zc_hbm, ones_hbm,
           sum_out, cnt0_out, cnt1_out, acc, cnt_sh, ones_v, *bufs):
    cid = lax.axis_index("c")
    sid = lax.axis_index("s")
    wid = cid * NS + sid
    base_n = sid * rows_per_sub
    base_c = wid * nchunks
    ivs, rvs, gs, ss, isems = (bufs[0:NB], bufs[NB:2 * NB],
                               bufs[2 * NB:3 * NB], bufs[3 * NB:4 * NB],
                               bufs[4 * NB:5 * NB])

    def start_idx(i, b):
      pltpu.async_copy(ei_hbm.at[base_c + i], ivs[b], isems[b])

    def wait_idx(i, b):
      pltpu.make_async_copy(ei_hbm.at[base_c + i], ivs[b], isems[b]).wait()

    def start_gather(b):
      pltpu.async_copy(x_hbm.at[ivs[b].at[0]], rvs[b], gs[b])

    def wait_gather(b):
      pltpu.make_async_copy(x_hbm.at[ivs[b].at[0]], rvs[b], gs[b]).wait()

    def start_scatter(b):
      pltpu.async_copy(rvs[b], acc.at[ivs[b].at[1]], ss[b], add=True)
      pltpu.async_copy(ones_v, cnt_sh.at[ivs[b].at[1]], ss[b], add=True)

    def wait_scatter(b):
      pltpu.make_async_copy(rvs[b], acc.at[ivs[b].at[1]], ss[b]).wait()
      pltpu.make_async_copy(ones_v, cnt_sh.at[ivs[b].at[1]], ss[b]).wait()

    pltpu.sync_copy(ones_hbm, ones_v)
    for b in range(NB - 1):
      start_idx(b, b)
    pltpu.sync_copy(zf_hbm, acc.at[pl.ds(base_n, rows_per_sub)])
    pltpu.sync_copy(zc_hbm, cnt_sh.at[pl.ds(base_n, rows_per_sub)])
    wait_idx(0, 0)
    start_gather(0)
    wait_idx(1, 1)
    start_gather(1)
    plsc.subcore_barrier()
    wait_gather(0)
    start_scatter(0)
    start_idx(3, 3)
    wait_idx(2, 2)
    start_gather(2)

    def slot(i, b):
      bn = (b + 1) % NB
      wait_scatter(bn)
      start_idx(i + 1, bn)
      wait_idx(i, b)
      start_gather(b)
      wait_gather((b - 2) % NB)
      start_scatter((b - 2) % NB)

    def slot_group(k, carry):
      i0 = 3 + NB * k
      for j in range(NB):
        slot(i0 + j, (3 + j) % NB)
      return carry

    lax.fori_loop(0, loop_iters, slot_group, 0)
    i = nchunks - 2
    b = i % NB
    wait_scatter((b + 1) % NB)
    start_idx(i + 1, (b + 1) % NB)
    wait_idx(i, b)
    start_gather(b)
    wait_gather((b - 2) % NB)
    start_scatter((b - 2) % NB)
    i = nchunks - 1
    b = i % NB
    wait_scatter((b + 1) % NB)
    wait_idx(i, b)
    start_gather(b)
    wait_gather((b - 2) % NB)
    start_scatter((b - 2) % NB)
    wait_gather((b - 1) % NB)
    start_scatter((b - 1) % NB)
    wait_gather(b)
    start_scatter(b)
    for bb in ((b - 2) % NB, (b - 1) % NB, b):
      wait_scatter(bb)
    plsc.subcore_barrier()
    pltpu.sync_copy(acc.at[pl.ds(base_n, rows_per_sub)],
                    sum_out.at[cid, pl.ds(base_n, rows_per_sub)])

    @pl.when(cid == 0)
    def _():
      pltpu.sync_copy(cnt_sh.at[pl.ds(base_n, rows_per_sub)],
                      cnt0_out.at[pl.ds(base_n, rows_per_sub)])

    @pl.when(cid == 1)
    def _():
      pltpu.sync_copy(cnt_sh.at[pl.ds(base_n, rows_per_sub)],
                      cnt1_out.at[pl.ds(base_n, rows_per_sub)])

  call = pl.kernel(
      body,
      out_type=(
          jax.ShapeDtypeStruct((NC, n_pad, d), jnp.float32),
          jax.ShapeDtypeStruct((n_pad, CW), jnp.float32),
          jax.ShapeDtypeStruct((n_pad, CW), jnp.float32),
      ),
      mesh=mesh,
      scratch_types=(
          [
              pltpu.VMEM_SHARED((n_pad, d), jnp.float32),
              pltpu.VMEM_SHARED((n_pad, CW), jnp.float32),
              pltpu.VMEM((C, CW), jnp.float32),
          ]
          + [pltpu.VMEM((2, C), jnp.int32) for _ in range(NB)]
          + [pltpu.VMEM((C, d), jnp.float32) for _ in range(NB)]
          + [pltpu.SemaphoreType.DMA for _ in range(3 * NB)]
      ),
      compiler_params=pltpu.CompilerParams(use_tc_tiling_on_sc=False),
  )
  return call(x, eiw, zeros_blk, zeros_cnt, ones_blk)


def _tc_mlp(x, sums, cnt0, cnt1, W_l, W_r, W1, W2, W3, b_l, b1, b2, b3):
  n, d = x.shape
  out_dim = W3.shape[0]
  R = 1000
  assert n % R == 0
  grid = n // R
  dn = (((1,), (1,)), ((), ()))

  def body(xb, s0b, s1b, c0b, c1b, wl, wr, w1, w2, w3,
           bl, bb1, bb2, bb3, ob):
    summed = s0b[0] + s1b[0]
    counts = c0b[:, :1] + c1b[:, :1]
    mean = summed / jnp.maximum(counts, 1.0)
    f32 = jnp.float32
    h = (lax.dot_general(mean, wl[...], dn, preferred_element_type=f32)
         + lax.dot_general(xb[...], wr[...], dn, preferred_element_type=f32)
         + bl[...])
    h1 = jnp.maximum(
        lax.dot_general(h, w1[...], dn, preferred_element_type=f32)
        + bb1[...], 0.0)
    h2 = jnp.maximum(
        lax.dot_general(h1, w2[...], dn, preferred_element_type=f32)
        + bb2[...], 0.0)
    ob[...] = (lax.dot_general(h2, w3[...], dn, preferred_element_type=f32)
               + bb3[...])

  row_spec = lambda c: pl.BlockSpec((R, c), lambda i: (i, 0))
  sum_spec = lambda k: pl.BlockSpec((1, R, d), lambda i, _k=k: (_k, i, 0))
  full_spec = lambda r, c: pl.BlockSpec((r, c), lambda i: (0, 0))
  return pl.pallas_call(
      body,
      grid=(grid,),
      in_specs=[
          row_spec(d), sum_spec(0), sum_spec(1), row_spec(CW), row_spec(CW),
          full_spec(*W_l.shape), full_spec(*W_r.shape),
          full_spec(*W1.shape), full_spec(*W2.shape), full_spec(*W3.shape),
          full_spec(*b_l.shape), full_spec(*b1.shape),
          full_spec(*b2.shape), full_spec(*b3.shape),
      ],
      out_specs=row_spec(out_dim),
      out_shape=jax.ShapeDtypeStruct((n, out_dim), jnp.float32),
  )(x, sums, sums, cnt0, cnt1, W_l, W_r, W1, W2, W3, b_l, b1, b2, b3)


@jax.jit
def kernel(x, edge_index, W_l, b_l, W_r, W1, b1, W2, b2, W3, b3):
  n, d = x.shape
  n_pad = ((n + NS * 8 - 1) // (NS * 8)) * (NS * 8)
  sums, cnt0, cnt1 = _sc_aggregate(x, edge_index, n_pad)
  return _tc_mlp(
      x, sums, cnt0, cnt1, W_l, W_r, W1, W2, W3,
      b_l.reshape(1, -1), b1.reshape(1, -1), b2.reshape(1, -1),
      b3.reshape(1, -1))
